# baseline (speedup 1.0000x reference)
import functools

import jax
import jax.numpy as jnp
from jax import lax
from jax.experimental import pallas as pl
from jax.experimental.pallas import tpu as pltpu
from jax.experimental.pallas import tpu_sc as plsc

N_NODES = 10000
NP = 10240
D_IN = 128
D_H = 256
G_SEG = 64

E_EDGES = 320000
E_EXT = E_EDGES + N_NODES
EPAD = 344064
PT = EPAD // 16
CHUNKS = PT // 128
ROWS_PT = NP // 16

V1 = 3456
V2 = 1792
CAP1 = 131072
CAP2 = 81920
T1 = CAP1 // 16 // 128
T2 = CAP2 // 16 // 128

_mesh = plsc.VectorSubcoreMesh(core_axis_name="c", subcore_axis_name="s")


@functools.partial(
    pl.kernel,
    out_type=jax.ShapeDtypeStruct((NP,), jnp.float32),
    mesh=_mesh,
    scratch_types=[
        pltpu.VMEM((CHUNKS, 128), jnp.int32),
        pltpu.VMEM((128,), jnp.float32),
        pltpu.VMEM((ROWS_PT,), jnp.float32),
        pltpu.VMEM_SHARED((NP,), jnp.float32),
    ],
)
def _deg(dst2, consts, deg_out, idx_d, ones_v, stage_v, deg_sp):
    cid = lax.axis_index("c")
    sid = lax.axis_index("s")
    pltpu.sync_copy(dst2.at[pl.ds(sid * CHUNKS, CHUNKS)], idx_d)
    pltpu.sync_copy(consts.at[pl.ds(ROWS_PT, 128)], ones_v)
    pltpu.sync_copy(consts.at[pl.ds(0, ROWS_PT)], stage_v)
    pltpu.sync_copy(stage_v, deg_sp.at[pl.ds(sid * ROWS_PT, ROWS_PT)])
    plsc.subcore_barrier()

    def body(c, _):
        pltpu.sync_copy(ones_v, deg_sp.at[idx_d.at[c]], add=True)
        return ()

    lax.fori_loop(0, CHUNKS, body, (), unroll=False)
    plsc.subcore_barrier()

    base = cid * (NP // 2) + sid * 320
    pltpu.sync_copy(deg_sp.at[pl.ds(base, 320)], stage_v.at[pl.ds(0, 320)])
    pltpu.sync_copy(stage_v.at[pl.ds(0, 320)], deg_out.at[pl.ds(base, 320)])


def _make_prop(vcap, tchunks, ib, drain_sizes):
    cap_rows = tchunks * 16 * 128
    nouter = tchunks // ib
    rows_t = vcap // 16

    @functools.partial(
        pl.kernel,
        out_type=jax.ShapeDtypeStruct((2 * vcap, 2, 128), jnp.float32),
        mesh=_mesh,
        scratch_types=[
            pltpu.VMEM((ib * 128,), jnp.int32),
            pltpu.VMEM((2 * ib, 128), jnp.int32),
            pltpu.VMEM((128, 2, 128), jnp.float32),
            pltpu.VMEM((128, 2, 128), jnp.float32),
            pltpu.VMEM_SHARED((2 * vcap, 128), jnp.float32),
            pltpu.SemaphoreType.DMA,
            pltpu.SemaphoreType.DMA,
            pltpu.SemaphoreType.DMA,
            pltpu.SemaphoreType.DMA,
            pltpu.SemaphoreType.DMA,
            pltpu.SemaphoreType.DMA,
        ],
    )
    def prop(g_hbm, srcp, dstp, zrows, out_hbm,
             idx_s, idx_d, buf_a, buf_b, acc_sp,
             sem_ga0, sem_ga1, sem_gb0, sem_gb1, sem_sa, sem_sb):
        cid = lax.axis_index("c")
        sid = lax.axis_index("s")
        pltpu.sync_copy(zrows, buf_a)
        off = 0
        for sz in drain_sizes:
            pltpu.sync_copy(
                buf_a.reshape(2 * 128, 128).at[pl.ds(0, 2 * sz)],
                acc_sp.at[pl.ds(2 * (sid * rows_t + off), 2 * sz)])
            off += sz
        plsc.subcore_barrier()

        def gather(c, buf, s0, s1):
            pltpu.async_copy(g_hbm.at[idx_s.at[pl.ds(c * 128, 64)]],
                             buf.at[pl.ds(0, 64)], s0)
            pltpu.async_copy(g_hbm.at[idx_s.at[pl.ds(c * 128 + 64, 64)]],
                             buf.at[pl.ds(64, 64)], s1)

        def scatter(c, buf, sem):
            hv = buf.reshape(2 * 128, 128)
            pltpu.async_copy(hv.at[pl.ds(0, 128)],
                             acc_sp.at[idx_d.at[2 * c]], sem, add=True)
            pltpu.async_copy(hv.at[pl.ds(128, 128)],
                             acc_sp.at[idx_d.at[2 * c + 1]], sem, add=True)

        def wait_gather(buf, s0, s1):
            pltpu.make_async_copy(g_hbm.at[pl.ds(0, 64)],
                                  buf.at[pl.ds(0, 64)], s0).wait()
            pltpu.make_async_copy(g_hbm.at[pl.ds(0, 64)],
                                  buf.at[pl.ds(64, 64)], s1).wait()

        def wait_scatter(c, buf, sem):
            hv = buf.reshape(2 * 128, 128)
            pltpu.make_async_copy(hv.at[pl.ds(0, 128)],
                                  acc_sp.at[idx_d.at[2 * c]], sem).wait()
            pltpu.make_async_copy(hv.at[pl.ds(128, 128)],
                                  acc_sp.at[idx_d.at[2 * c + 1]], sem).wait()

        def outer(ob, _):
            pltpu.sync_copy(
                srcp.at[pl.ds(cid * cap_rows + sid * (tchunks * 128)
                              + ob * (ib * 128), ib * 128)], idx_s)
            pltpu.sync_copy(
                dstp.at[pl.ds(2 * (cid * (cap_rows // 128) + sid * tchunks
                                   + ob * ib), 2 * ib)], idx_d)
            gather(0, buf_a, sem_ga0, sem_ga1)
            gather(1, buf_b, sem_gb0, sem_gb1)

            def pair(p, _):
                for c, buf, sg0, sg1, ss in (
                        (2 * p, buf_a, sem_ga0, sem_ga1, sem_sa),
                        (2 * p + 1, buf_b, sem_gb0, sem_gb1, sem_sb)):
                    wait_gather(buf, sg0, sg1)
                    scatter(c, buf, ss)

                    @pl.when(p < ib // 2 - 1)
                    def _():
                        wait_scatter(c, buf, ss)
                        gather(c + 2, buf, sg0, sg1)

                return ()

            lax.fori_loop(0, ib // 2, pair, (), unroll=False)
            wait_scatter(ib - 2, buf_a, sem_sa)
            wait_scatter(ib - 1, buf_b, sem_sb)
            return ()

        lax.fori_loop(0, nouter, outer, (), unroll=False)
        plsc.subcore_barrier()

        off = 0
        for sz in drain_sizes:
            pltpu.sync_copy(
                acc_sp.at[pl.ds(2 * (sid * rows_t + off), 2 * sz)],
                buf_a.reshape(2 * 128, 128).at[pl.ds(0, 2 * sz)])
            pltpu.sync_copy(buf_a.at[pl.ds(0, sz)],
                            out_hbm.at[pl.ds(cid * vcap + sid * rows_t + off,
                                             sz)])
            off += sz

    return prop


_prop1 = _make_prop(V1, T1, 16, (128, 88))
_prop2 = _make_prop(V2, T2, 8, (112,))


def _dinv(deg_ref):
    return lax.rsqrt(jnp.maximum(deg_ref[...], 1.0))


def _mm1_body(x_ref, w_ref, deg_ref, o_ref):
    h = jnp.dot(x_ref[...], w_ref[...], preferred_element_type=jnp.float32)
    o_ref[...] = _dinv(deg_ref) * h


def _mm1(xp, W1, deg_col):
    mb = 512
    return pl.pallas_call(
        _mm1_body,
        grid=(NP // mb,),
        in_specs=[
            pl.BlockSpec((mb, D_IN), lambda m: (m, 0)),
            pl.BlockSpec((D_IN, D_H), lambda m: (0, 0)),
            pl.BlockSpec((mb, 1), lambda m: (m, 0)),
        ],
        out_specs=pl.BlockSpec((mb, D_H), lambda m: (m, 0)),
        out_shape=jax.ShapeDtypeStruct((NP, D_H), jnp.float32),
    )(xp, W1, deg_col)


def _mm23_body(a_ref, deg_ref, b_ref, w_ref, o_ref):
    dinv = _dinv(deg_ref)
    z = jnp.maximum(dinv * a_ref[...] + b_ref[...], 0.0)
    h = jnp.dot(z, w_ref[...], preferred_element_type=jnp.float32)
    o_ref[...] = dinv * h


def _mm23(acc, deg_col, b_row, W):
    mb = 512
    return pl.pallas_call(
        _mm23_body,
        grid=(NP // mb,),
        in_specs=[
            pl.BlockSpec((mb, D_H), lambda m: (m, 0)),
            pl.BlockSpec((mb, 1), lambda m: (m, 0)),
            pl.BlockSpec((1, D_H), lambda m: (0, 0)),
            pl.BlockSpec((D_H, D_H), lambda m: (0, 0)),
        ],
        out_specs=pl.BlockSpec((mb, D_H), lambda m: (m, 0)),
        out_shape=jax.ShapeDtypeStruct((NP, D_H), jnp.float32),
    )(acc, deg_col, b_row, W)


def _pool_body(a_ref, deg_ref, b_ref, batch_ref, wp_ref, bp_ref,
               o_ref, pooled_acc, cnt_acc):
    m = pl.program_id(0)

    @pl.when(m == 0)
    def _():
        pooled_acc[...] = jnp.zeros_like(pooled_acc)
        cnt_acc[...] = jnp.zeros_like(cnt_acc)

    z = _dinv(deg_ref) * a_ref[...] + b_ref[...]
    ids = lax.broadcasted_iota(jnp.int32, (1, G_SEG), 1)
    oh = (batch_ref[...] == ids).astype(jnp.float32)
    contract = (((0,), (0,)), ((), ()))
    pooled_acc[...] += lax.dot_general(
        oh, z, contract, preferred_element_type=jnp.float32)
    ones = jnp.ones((oh.shape[0], 1), jnp.float32)
    cnt_acc[...] += lax.dot_general(
        oh, ones, contract, preferred_element_type=jnp.float32)

    pooled = pooled_acc[...] / jnp.maximum(cnt_acc[...], 1.0)
    o_ref[...] = jnp.dot(pooled, wp_ref[...],
                         preferred_element_type=jnp.float32) + bp_ref[...]


def _pool(acc, deg_col, b_row, batch2d, Wp, bp2d):
    mb = 1024
    return pl.pallas_call(
        _pool_body,
        grid=(NP // mb,),
        in_specs=[
            pl.BlockSpec((mb, D_H), lambda m: (m, 0)),
            pl.BlockSpec((mb, 1), lambda m: (m, 0)),
            pl.BlockSpec((1, D_H), lambda m: (0, 0)),
            pl.BlockSpec((mb, 1), lambda m: (m, 0)),
            pl.BlockSpec((D_H, 1), lambda m: (0, 0)),
            pl.BlockSpec((1, 1), lambda m: (0, 0)),
        ],
        out_specs=pl.BlockSpec((G_SEG, 1), lambda m: (0, 0)),
        out_shape=jax.ShapeDtypeStruct((G_SEG, 1), jnp.float32),
        scratch_shapes=[
            pltpu.VMEM((G_SEG, D_H), jnp.float32),
            pltpu.VMEM((G_SEG, 1), jnp.float32),
        ],
    )(acc, deg_col, b_row, batch2d, Wp, bp2d)


@jax.jit
def kernel(x, edge_index, batch, W1, b1, W2, b2, W3, b3, Wp, bp):
    src = edge_index[0]
    dst = edge_index[1]
    loop = jnp.arange(N_NODES, dtype=jnp.int32)
    src_e = jnp.concatenate([src, loop])
    dst_e = jnp.concatenate([dst, loop])

    padi = jnp.full((EPAD - E_EXT,), NP - 1, dtype=jnp.int32)
    dst2 = jnp.concatenate([dst_e, padi]).reshape(EPAD // 128, 128)

    th = jnp.array([V1, 2 * V1, 2 * V1 + V2], jnp.int32)
    cat = ((dst_e >= th[0]).astype(jnp.int32)
           + (dst_e >= th[1]).astype(jnp.int32)
           + (dst_e >= th[2]).astype(jnp.int32))
    oh4 = (cat[:, None] == jnp.arange(4, dtype=jnp.int32)[None, :])
    pc = jnp.cumsum(oh4.astype(jnp.int32), axis=0)
    pic = jnp.take_along_axis(pc, cat[:, None], axis=1)[:, 0] - 1
    caps = jnp.array([CAP1, CAP1, CAP2, CAP2], jnp.int32)
    basesa = jnp.array([0, CAP1, 2 * CAP1, 2 * CAP1 + CAP2], jnp.int32)
    dlb = jnp.array([0, V1, 2 * V1, 2 * V1 + V2], jnp.int32)
    pos = jnp.where(pic < caps[cat], basesa[cat] + pic, 1 << 30)
    dst_loc = dst_e - dlb[cat]
    tot = 2 * CAP1 + 2 * CAP2
    srcp = jnp.full((tot,), NP - 1, jnp.int32).at[pos].set(
        src_e, mode="drop", unique_indices=True)
    dstp = jnp.zeros((tot,), jnp.int32).at[pos].set(
        dst_loc, mode="drop", unique_indices=True)
    srcp1 = srcp[:2 * CAP1]
    srcp2 = srcp[2 * CAP1:]
    dsti = (dstp[:, None] * 2
            + jnp.arange(2, dtype=jnp.int32)[None, :]).reshape(2 * tot)
    dstp1 = dsti[:4 * CAP1].reshape(4 * CAP1 // 128, 128)
    dstp2 = dsti[4 * CAP1:].reshape(4 * CAP2 // 128, 128)

    xp = jnp.zeros((NP, D_IN), jnp.float32).at[:N_NODES].set(x)
    batch2d = jnp.concatenate(
        [batch, jnp.full((NP - N_NODES,), G_SEG, jnp.int32)]).reshape(NP, 1)
    consts = jnp.concatenate(
        [jnp.zeros((ROWS_PT,), jnp.float32), jnp.ones((128,), jnp.float32)])
    zrows = jnp.zeros((128, 2, 128), jnp.float32)

    deg = _deg(dst2, consts)
    deg_col = deg.reshape(NP, 1)

    def prop_all(g):
        g3 = g.reshape(NP, 2, 128)
        o1 = _prop1(g3, srcp1, dstp1, zrows)
        o2 = _prop2(g3, srcp2, dstp2, zrows)
        return jnp.concatenate([o1, o2]).reshape(2 * (V1 + V2), D_H)

    g = _mm1(xp, W1, deg_col)
    acc = prop_all(g)
    g = _mm23(acc, deg_col, b1.reshape(1, D_H), W2)
    acc = prop_all(g)
    g = _mm23(acc, deg_col, b2.reshape(1, D_H), W3)
    acc = prop_all(g)
    return _pool(acc, deg_col, b3.reshape(1, D_H), batch2d, Wp,
                 bp.reshape(1, 1))


# speedup vs baseline: 5.3178x; 5.3178x over previous
import functools

import jax
import jax.numpy as jnp
from jax import lax
from jax.experimental import pallas as pl
from jax.experimental.pallas import tpu as pltpu
from jax.experimental.pallas import tpu_sc as plsc

N_NODES = 10000
NP = 10240
D_IN = 128
D_H = 256
HALF = 128
G_SEG = 64

E_EDGES = 320000
E_EXT = E_EDGES + N_NODES
EPAD = 344064
PT = EPAD // 16
CHUNKS = PT // 128
ROWS_PT = NP // 16

_mesh = plsc.VectorSubcoreMesh(core_axis_name="c", subcore_axis_name="s")


@functools.partial(
    pl.kernel,
    out_type=jax.ShapeDtypeStruct((NP,), jnp.float32),
    mesh=_mesh,
    scratch_types=[
        pltpu.VMEM((CHUNKS, 128), jnp.int32),
        pltpu.VMEM((128,), jnp.float32),
        pltpu.VMEM((ROWS_PT,), jnp.float32),
        pltpu.VMEM_SHARED((NP,), jnp.float32),
    ],
)
def _deg(dst2, consts, deg_out, idx_d, ones_v, stage_v, deg_sp):
    cid = lax.axis_index("c")
    sid = lax.axis_index("s")
    pltpu.sync_copy(dst2.at[pl.ds(sid * CHUNKS, CHUNKS)], idx_d)
    pltpu.sync_copy(consts.at[pl.ds(ROWS_PT, 128)], ones_v)
    pltpu.sync_copy(consts.at[pl.ds(0, ROWS_PT)], stage_v)
    pltpu.sync_copy(stage_v, deg_sp.at[pl.ds(sid * ROWS_PT, ROWS_PT)])
    plsc.subcore_barrier()

    def body(c, _):
        pltpu.sync_copy(ones_v, deg_sp.at[idx_d.at[c]], add=True)
        return ()

    lax.fori_loop(0, CHUNKS, body, (), unroll=False)
    plsc.subcore_barrier()

    base = cid * (NP // 2) + sid * 320
    pltpu.sync_copy(deg_sp.at[pl.ds(base, 320)], stage_v.at[pl.ds(0, 320)])
    pltpu.sync_copy(stage_v.at[pl.ds(0, 320)], deg_out.at[pl.ds(base, 320)])


IB = 56
OUTER = CHUNKS // IB


@functools.partial(
    pl.kernel,
    out_type=jax.ShapeDtypeStruct((2 * NP, HALF), jnp.float32),
    mesh=_mesh,
    scratch_types=[
        pltpu.VMEM((IB * 128,), jnp.int32),
        pltpu.VMEM((IB, 128), jnp.int32),
        pltpu.VMEM((128, HALF), jnp.float32),
        pltpu.VMEM((128, HALF), jnp.float32),
        pltpu.VMEM_SHARED((NP, HALF), jnp.float32),
        pltpu.SemaphoreType.DMA,
        pltpu.SemaphoreType.DMA,
        pltpu.SemaphoreType.DMA,
        pltpu.SemaphoreType.DMA,
        pltpu.SemaphoreType.DMA,
        pltpu.SemaphoreType.DMA,
    ],
)
def _propagate(g_hbm, src2, dst2, zrows, out_hbm,
               idx_s, idx_d, buf_a, buf_b, acc_sp,
               sem_ga0, sem_ga1, sem_gb0, sem_gb1, sem_sa, sem_sb):
    cid = lax.axis_index("c")
    sid = lax.axis_index("s")
    pltpu.sync_copy(zrows, buf_a)
    for k in range(ROWS_PT // 128):
        pltpu.sync_copy(buf_a, acc_sp.at[pl.ds(sid * ROWS_PT + k * 128, 128)])
    plsc.subcore_barrier()

    def gather(c, buf, s0, s1):
        pltpu.async_copy(g_hbm.at[idx_s.at[pl.ds(c * 128, 64)]],
                         buf.at[pl.ds(0, 64)], s0)
        pltpu.async_copy(g_hbm.at[idx_s.at[pl.ds(c * 128 + 64, 64)]],
                         buf.at[pl.ds(64, 64)], s1)

    def scatter(c, buf, sem):
        return pltpu.async_copy(buf, acc_sp.at[idx_d.at[c]], sem, add=True)

    def wait_gather(buf, s0, s1):
        pltpu.make_async_copy(g_hbm.at[pl.ds(0, 64)],
                              buf.at[pl.ds(0, 64)], s0).wait()
        pltpu.make_async_copy(g_hbm.at[pl.ds(0, 64)],
                              buf.at[pl.ds(64, 64)], s1).wait()

    def wait_scatter(c, buf, sem):
        pltpu.make_async_copy(buf, acc_sp.at[idx_d.at[c]], sem).wait()

    def outer(ob, _):
        pltpu.sync_copy(
            src2.at[pl.ds(cid * EPAD + sid * PT + ob * (IB * 128), IB * 128)],
            idx_s)
        pltpu.sync_copy(dst2.at[pl.ds(sid * CHUNKS + ob * IB, IB)], idx_d)
        gather(0, buf_a, sem_ga0, sem_ga1)
        gather(1, buf_b, sem_gb0, sem_gb1)

        def pair(p, _):
            for c, buf, sg0, sg1, ss in (
                    (2 * p, buf_a, sem_ga0, sem_ga1, sem_sa),
                    (2 * p + 1, buf_b, sem_gb0, sem_gb1, sem_sb)):
                wait_gather(buf, sg0, sg1)
                scatter(c, buf, ss)

                @pl.when(p < IB // 2 - 1)
                def _():
                    wait_scatter(c, buf, ss)
                    gather(c + 2, buf, sg0, sg1)

            return ()

        lax.fori_loop(0, IB // 2, pair, (), unroll=False)
        wait_scatter(IB - 2, buf_a, sem_sa)
        wait_scatter(IB - 1, buf_b, sem_sb)
        return ()

    lax.fori_loop(0, OUTER, outer, (), unroll=False)
    plsc.subcore_barrier()

    base = sid * ROWS_PT
    for k in range(ROWS_PT // 128):
        pltpu.sync_copy(acc_sp.at[pl.ds(base + k * 128, 128)], buf_a)
        pltpu.sync_copy(buf_a, out_hbm.at[pl.ds(cid * NP + base + k * 128,
                                                128)])


def _dinv(deg_ref):
    return lax.rsqrt(jnp.maximum(deg_ref[...], 1.0))


def _mm1_body(x_ref, w_ref, deg_ref, o_ref):
    h = jnp.dot(x_ref[...], w_ref[...], preferred_element_type=jnp.float32)
    o_ref[...] = _dinv(deg_ref) * h


def _mm1(xp, W1, deg_col):
    mb = 512
    return pl.pallas_call(
        _mm1_body,
        grid=(NP // mb, 2),
        in_specs=[
            pl.BlockSpec((mb, D_IN), lambda m, j: (m, 0)),
            pl.BlockSpec((D_IN, HALF), lambda m, j: (0, j)),
            pl.BlockSpec((mb, 1), lambda m, j: (m, 0)),
        ],
        out_specs=pl.BlockSpec((mb, HALF), lambda m, j: (m + j * (NP // mb), 0)),
        out_shape=jax.ShapeDtypeStruct((2 * NP, HALF), jnp.float32),
    )(xp, W1, deg_col)


def _mm23_body(a0_ref, a1_ref, deg_ref, b_ref, w_ref, o_ref):
    dinv = _dinv(deg_ref)
    z0 = jnp.maximum(dinv * a0_ref[...] + b_ref[:, :HALF], 0.0)
    z1 = jnp.maximum(dinv * a1_ref[...] + b_ref[:, HALF:], 0.0)
    h = (jnp.dot(z0, w_ref[:HALF, :], preferred_element_type=jnp.float32)
         + jnp.dot(z1, w_ref[HALF:, :], preferred_element_type=jnp.float32))
    o_ref[...] = dinv * h


def _mm23(acc, deg_col, b_row, W):
    mb = 512
    return pl.pallas_call(
        _mm23_body,
        grid=(NP // mb, 2),
        in_specs=[
            pl.BlockSpec((mb, HALF), lambda m, j: (m, 0)),
            pl.BlockSpec((mb, HALF), lambda m, j: (m + NP // mb, 0)),
            pl.BlockSpec((mb, 1), lambda m, j: (m, 0)),
            pl.BlockSpec((1, D_H), lambda m, j: (0, 0)),
            pl.BlockSpec((D_H, HALF), lambda m, j: (0, j)),
        ],
        out_specs=pl.BlockSpec((mb, HALF), lambda m, j: (m + j * (NP // mb), 0)),
        out_shape=jax.ShapeDtypeStruct((2 * NP, HALF), jnp.float32),
    )(acc, acc, deg_col, b_row, W)


def _pool_body(a0_ref, a1_ref, deg_ref, b_ref, batch_ref, wp_ref, bp_ref,
               o_ref, pooled_acc, cnt_acc):
    m = pl.program_id(0)

    @pl.when(m == 0)
    def _():
        pooled_acc[...] = jnp.zeros_like(pooled_acc)
        cnt_acc[...] = jnp.zeros_like(cnt_acc)

    dinv = _dinv(deg_ref)
    z0 = dinv * a0_ref[...] + b_ref[:, :HALF]
    z1 = dinv * a1_ref[...] + b_ref[:, HALF:]
    ids = lax.broadcasted_iota(jnp.int32, (1, G_SEG), 1)
    oh = (batch_ref[...] == ids).astype(jnp.float32)
    contract = (((0,), (0,)), ((), ()))
    pooled_acc[:, :HALF] += lax.dot_general(
        oh, z0, contract, preferred_element_type=jnp.float32)
    pooled_acc[:, HALF:] += lax.dot_general(
        oh, z1, contract, preferred_element_type=jnp.float32)
    ones = jnp.ones((oh.shape[0], 1), jnp.float32)
    cnt_acc[...] += lax.dot_general(
        oh, ones, contract, preferred_element_type=jnp.float32)

    pooled = pooled_acc[...] / jnp.maximum(cnt_acc[...], 1.0)
    o_ref[...] = jnp.dot(pooled, wp_ref[...],
                         preferred_element_type=jnp.float32) + bp_ref[...]


def _pool(acc, deg_col, b_row, batch2d, Wp, bp2d):
    mb = 1024
    return pl.pallas_call(
        _pool_body,
        grid=(NP // mb,),
        in_specs=[
            pl.BlockSpec((mb, HALF), lambda m: (m, 0)),
            pl.BlockSpec((mb, HALF), lambda m: (m + NP // mb, 0)),
            pl.BlockSpec((mb, 1), lambda m: (m, 0)),
            pl.BlockSpec((1, D_H), lambda m: (0, 0)),
            pl.BlockSpec((mb, 1), lambda m: (m, 0)),
            pl.BlockSpec((D_H, 1), lambda m: (0, 0)),
            pl.BlockSpec((1, 1), lambda m: (0, 0)),
        ],
        out_specs=pl.BlockSpec((G_SEG, 1), lambda m: (0, 0)),
        out_shape=jax.ShapeDtypeStruct((G_SEG, 1), jnp.float32),
        scratch_shapes=[
            pltpu.VMEM((G_SEG, D_H), jnp.float32),
            pltpu.VMEM((G_SEG, 1), jnp.float32),
        ],
    )(acc, acc, deg_col, b_row, batch2d, Wp, bp2d)


@jax.jit
def kernel(x, edge_index, batch, W1, b1, W2, b2, W3, b3, Wp, bp):
    src = edge_index[0]
    dst = edge_index[1]
    loop = jnp.arange(N_NODES, dtype=jnp.int32)
    padi = jnp.full((EPAD - E_EXT,), NP - 1, dtype=jnp.int32)
    src_ext = jnp.concatenate([src, loop, padi])
    dst_ext = jnp.concatenate([dst, loop, padi])
    src2 = jnp.concatenate([src_ext, src_ext + NP])
    dst2 = dst_ext.reshape(EPAD // 128, 128)

    xp = jnp.zeros((NP, D_IN), jnp.float32).at[:N_NODES].set(x)
    batch2d = jnp.concatenate(
        [batch, jnp.full((NP - N_NODES,), G_SEG, jnp.int32)]).reshape(NP, 1)
    consts = jnp.concatenate(
        [jnp.zeros((ROWS_PT,), jnp.float32), jnp.ones((128,), jnp.float32)])
    zrows = jnp.zeros((128, HALF), jnp.float32)

    deg = _deg(dst2, consts)
    deg_col = deg.reshape(NP, 1)

    g = _mm1(xp, W1, deg_col)
    acc = _propagate(g, src2, dst2, zrows)
    g = _mm23(acc, deg_col, b1.reshape(1, D_H), W2)
    acc = _propagate(g, src2, dst2, zrows)
    g = _mm23(acc, deg_col, b2.reshape(1, D_H), W3)
    acc = _propagate(g, src2, dst2, zrows)
    return _pool(acc, deg_col, b3.reshape(1, D_H), batch2d, Wp,
                 bp.reshape(1, 1))
